# baseline (speedup 1.0000x reference)
import jax
import jax.numpy as jnp
from jax import lax
from jax.experimental import pallas as pl
from jax.experimental.pallas import tpu as pltpu
from jax.experimental.pallas import tpu_sc as plsc

N_NODES = 10000
N_EDGES = 5000
NNZ = 320000
D = 128

NC = 2
NS = 16
NW = NC * NS
CHUNK = 128
CPW = 80
G = 16
NG = CPW // G
NNZ_PAD = NW * CPW * CHUNK
NPAD = 10240
GARBAGE = NPAD - 1

ROWS_BLK = 1000


def _mesh():
    return plsc.VectorSubcoreMesh(core_axis_name="c", subcore_axis_name="s")


def _segsum_body(tbl, sidx, didx, out,
                 sidx_v, didx_v, rows0, rows1, acc, sem0, sem1):
    cid = lax.axis_index("c")
    sid = lax.axis_index("s")
    wid = cid * NS + sid
    npr = NPAD // NS

    def zfill(i, _):
        for k in range(D // 16):
            rows0[i, pl.ds(16 * k, 16)] = jnp.zeros((16,), jnp.float32)
        return 0
    lax.fori_loop(0, CHUNK, zfill, 0)
    base = sid * npr
    for t in range(npr // CHUNK):
        pltpu.sync_copy(rows0, acc.at[pl.ds(base + t * CHUNK, CHUNK)])
    plsc.subcore_barrier()

    def group(g, _):
        pltpu.sync_copy(sidx.at[wid].at[pl.ds(g * G, G)], sidx_v)
        pltpu.sync_copy(didx.at[wid].at[pl.ds(g * G, G)], didx_v)

        pltpu.async_copy(tbl.at[sidx_v.at[0]], rows0, sem0)
        pltpu.async_copy(tbl.at[sidx_v.at[1]], rows1, sem1)

        def body(i, _):
            j0 = 2 * i
            pltpu.make_async_copy(tbl.at[sidx_v.at[j0]], rows0, sem0).wait()
            pltpu.sync_copy(rows0, acc.at[didx_v.at[j0]], add=True)

            @pl.when(i < G // 2 - 1)
            def _():
                pltpu.async_copy(tbl.at[sidx_v.at[j0 + 2]], rows0, sem0)

            pltpu.make_async_copy(tbl.at[sidx_v.at[j0 + 1]], rows1, sem1).wait()
            pltpu.sync_copy(rows1, acc.at[didx_v.at[j0 + 1]], add=True)

            @pl.when(i < G // 2 - 1)
            def _():
                pltpu.async_copy(tbl.at[sidx_v.at[j0 + 3]], rows1, sem1)
            return 0
        lax.fori_loop(0, G // 2, body, 0)
        return 0
    lax.fori_loop(0, NG, group, 0)
    plsc.subcore_barrier()

    pltpu.sync_copy(acc.at[pl.ds(base, npr)],
                    out.at[cid].at[pl.ds(base, npr)])


_segsum = pl.kernel(
    _segsum_body,
    out_type=jax.ShapeDtypeStruct((NC, NPAD, D), jnp.float32),
    mesh=_mesh(),
    scratch_types=[
        pltpu.VMEM((G, CHUNK), jnp.int32),
        pltpu.VMEM((G, CHUNK), jnp.int32),
        pltpu.VMEM((CHUNK, D), jnp.float32),
        pltpu.VMEM((CHUNK, D), jnp.float32),
        pltpu.VMEM_SHARED((NPAD, D), jnp.float32),
        pltpu.SemaphoreType.DMA,
        pltpu.SemaphoreType.DMA,
    ],
)


def _ones_scatter_body(didx, out, didx_v, rows0, acc):
    cid = lax.axis_index("c")
    sid = lax.axis_index("s")
    wid = cid * NS + sid
    npr = NPAD // NS

    def zfill(i, _):
        for k in range(D // 16):
            rows0[i, pl.ds(16 * k, 16)] = jnp.zeros((16,), jnp.float32)
        return 0
    lax.fori_loop(0, CHUNK, zfill, 0)
    base = sid * npr
    for t in range(npr // CHUNK):
        pltpu.sync_copy(rows0, acc.at[pl.ds(base + t * CHUNK, CHUNK)])

    def ofill(i, _):
        for k in range(D // 16):
            rows0[i, pl.ds(16 * k, 16)] = jnp.ones((16,), jnp.float32)
        return 0
    lax.fori_loop(0, CHUNK, ofill, 0)
    plsc.subcore_barrier()

    def group(g, _):
        pltpu.sync_copy(didx.at[wid].at[pl.ds(g * G, G)], didx_v)

        def body(j, _):
            pltpu.sync_copy(rows0, acc.at[didx_v.at[j]], add=True)
            return 0
        lax.fori_loop(0, G, body, 0)
        return 0
    lax.fori_loop(0, NG, group, 0)
    plsc.subcore_barrier()

    pltpu.sync_copy(acc.at[pl.ds(base, npr)],
                    out.at[cid].at[pl.ds(base, npr)])


_ones_scatter = pl.kernel(
    _ones_scatter_body,
    out_type=jax.ShapeDtypeStruct((NC, NPAD, D), jnp.float32),
    mesh=_mesh(),
    scratch_types=[
        pltpu.VMEM((G, CHUNK), jnp.int32),
        pltpu.VMEM((CHUNK, D), jnp.float32),
        pltpu.VMEM_SHARED((NPAD, D), jnp.float32),
    ],
)


def _dv_scale(dvp):
    cnt = dvp[0, :, 0] + dvp[1, :, 0]
    return jnp.where(cnt > 0, lax.rsqrt(cnt), 0.0)


def _de_scale(dep):
    cnt = dep[0, :, 0] + dep[1, :, 0]
    return jnp.where(cnt > 0, 1.0 / cnt, 0.0)


def _lrelu(x):
    return jnp.where(x >= 0, x, 0.1 * x)


def _stage_a_kernel(x_ref, w_ref, b_ref, dvp_ref, xs_ref, skip_ref):
    y = lax.dot_general(x_ref[...], w_ref[...], (((1,), (0,)), ((), ())),
                        preferred_element_type=jnp.float32) + b_ref[...]
    scale = _dv_scale(dvp_ref[...])
    xs_ref[...] = y[:, :D] * scale[:, None]
    skip_ref[...] = y[:, D:]


def _stage_a(x, w_cat, b_cat, dvp):
    nb = N_NODES // ROWS_BLK
    return pl.pallas_call(
        _stage_a_kernel,
        grid=(nb,),
        in_specs=[
            pl.BlockSpec((ROWS_BLK, D), lambda i: (i, 0)),
            pl.BlockSpec((D, 2 * D), lambda i: (0, 0)),
            pl.BlockSpec((1, 2 * D), lambda i: (0, 0)),
            pl.BlockSpec((NC, ROWS_BLK, D), lambda i: (0, i, 0)),
        ],
        out_specs=[pl.BlockSpec((ROWS_BLK, D), lambda i: (i, 0)),
                   pl.BlockSpec((ROWS_BLK, D), lambda i: (i, 0))],
        out_shape=[jax.ShapeDtypeStruct((NPAD, D), jnp.float32),
                   jax.ShapeDtypeStruct((N_NODES, D), jnp.float32)],
    )(x, w_cat, b_cat, dvp)


def _stage_b_kernel(ep_ref, dep_ref, eout_ref):
    de = _de_scale(dep_ref[...])[:, None]
    e = (ep_ref[0] + ep_ref[1]) * de
    eout_ref[...] = _lrelu(e) * de


def _stage_b(ep, dep):
    nb = N_EDGES // ROWS_BLK
    return pl.pallas_call(
        _stage_b_kernel,
        grid=(nb,),
        in_specs=[
            pl.BlockSpec((NC, ROWS_BLK, D), lambda i: (0, i, 0)),
            pl.BlockSpec((NC, ROWS_BLK, D), lambda i: (0, i, 0)),
        ],
        out_specs=pl.BlockSpec((ROWS_BLK, D), lambda i: (i, 0)),
        out_shape=jax.ShapeDtypeStruct((NPAD, D), jnp.float32),
    )(ep, dep)


def _stage_c_kernel(xp_ref, skip_ref, dvp_ref, xout_ref):
    dv = _dv_scale(dvp_ref[...])[:, None]
    xn = (xp_ref[0] + xp_ref[1]) * dv + skip_ref[...]
    xout_ref[...] = _lrelu(xn)


def _stage_c(xp, skip, dvp):
    nb = N_NODES // ROWS_BLK
    return pl.pallas_call(
        _stage_c_kernel,
        grid=(nb,),
        in_specs=[
            pl.BlockSpec((NC, ROWS_BLK, D), lambda i: (0, i, 0)),
            pl.BlockSpec((ROWS_BLK, D), lambda i: (i, 0)),
            pl.BlockSpec((NC, ROWS_BLK, D), lambda i: (0, i, 0)),
        ],
        out_specs=pl.BlockSpec((ROWS_BLK, D), lambda i: (i, 0)),
        out_shape=jax.ShapeDtypeStruct((N_NODES, D), jnp.float32),
    )(xp, skip, dvp)


def _scale_in_kernel(x_ref, dvp_ref, out_ref):
    out_ref[...] = x_ref[...] * _dv_scale(dvp_ref[...])[:, None]


def _scale_in(x, dvp):
    nb = N_NODES // ROWS_BLK
    return pl.pallas_call(
        _scale_in_kernel,
        grid=(nb,),
        in_specs=[
            pl.BlockSpec((ROWS_BLK, D), lambda i: (i, 0)),
            pl.BlockSpec((NC, ROWS_BLK, D), lambda i: (0, i, 0)),
        ],
        out_specs=pl.BlockSpec((ROWS_BLK, D), lambda i: (i, 0)),
        out_shape=jax.ShapeDtypeStruct((NPAD, D), jnp.float32),
    )(x, dvp)


def _final_e_kernel(ep_ref, dep_ref, out_ref):
    out_ref[...] = (ep_ref[0] + ep_ref[1]) * _de_scale(dep_ref[...])[:, None]


def _final_e(ep, dep):
    nb = N_EDGES // ROWS_BLK
    return pl.pallas_call(
        _final_e_kernel,
        grid=(nb,),
        in_specs=[
            pl.BlockSpec((NC, ROWS_BLK, D), lambda i: (0, i, 0)),
            pl.BlockSpec((NC, ROWS_BLK, D), lambda i: (0, i, 0)),
        ],
        out_specs=pl.BlockSpec((ROWS_BLK, D), lambda i: (i, 0)),
        out_shape=jax.ShapeDtypeStruct((N_EDGES, D), jnp.float32),
    )(ep, dep)


@jax.jit
def kernel(X, node_idx, edge_idx, params):
    pad = NNZ_PAD - NNZ
    shape3 = (NW, CPW, CHUNK)
    zpad = jnp.zeros((pad,), jnp.int32)
    gpad = jnp.full((pad,), GARBAGE, jnp.int32)
    ordn = jnp.argsort(node_idx)
    orde = jnp.argsort(edge_idx)
    nidx_src = jnp.concatenate([node_idx[ordn], zpad]).reshape(shape3)
    eidx_srt_dst = jnp.concatenate([edge_idx[ordn], gpad]).reshape(shape3)
    eidx_src = jnp.concatenate([edge_idx[orde], zpad]).reshape(shape3)
    nidx_srt_dst = jnp.concatenate([node_idx[orde], gpad]).reshape(shape3)
    nidx_dst = jnp.concatenate([node_idx, gpad]).reshape(shape3)
    eidx_dst = jnp.concatenate([edge_idx, gpad]).reshape(shape3)

    dvp = _ones_scatter(nidx_dst)
    dep = _ones_scatter(eidx_dst)

    for layer in params:
        for wkey, bkey, pkey, pbkey in (("fc1_w", "fc1_b", "proj1_w", "proj1_b"),
                                        ("fc2_w", "fc2_b", "proj2_w", "proj2_b")):
            w_cat = jnp.concatenate(
                [layer[wkey].T, layer[pkey].T], axis=1)
            b_cat = jnp.concatenate(
                [layer[bkey], layer[pbkey]]).reshape(1, 2 * D)
            xs, skip = _stage_a(X, w_cat, b_cat, dvp)
            ep = _segsum(xs, nidx_src, eidx_srt_dst)
            ein = _stage_b(ep, dep)
            xp = _segsum(ein, eidx_src, nidx_srt_dst)
            X = _stage_c(xp, skip, dvp)

    xs_f = _scale_in(X, dvp)
    ep_f = _segsum(xs_f, nidx_src, eidx_srt_dst)
    e_final = _final_e(ep_f, dep)
    return (e_final, X)


# speedup vs baseline: 1.2623x; 1.2623x over previous
import jax
import jax.numpy as jnp
from jax import lax
from jax.experimental import pallas as pl
from jax.experimental.pallas import tpu as pltpu
from jax.experimental.pallas import tpu_sc as plsc

N_NODES = 10000
N_EDGES = 5000
NNZ = 320000
D = 128

NC = 2
NS = 16
NW = NC * NS
CHUNK = 128
CPW = 80
G = 16
NG = CPW // G
NNZ_PAD = NW * CPW * CHUNK
NPAD = 10240
GARBAGE = NPAD - 1

SPLIT = 4
HS = CHUNK // SPLIT

ROWS_BLK = 1000


def _mesh():
    return plsc.VectorSubcoreMesh(core_axis_name="c", subcore_axis_name="s")


def _segsum_body(tbl, sidx, didx, out,
                 sidx_v, didx_v, rows0, rows1, acc, sem0, sem1):
    cid = lax.axis_index("c")
    sid = lax.axis_index("s")
    wid = cid * NS + sid
    npr = NPAD // NS

    def zfill(i, _):
        for k in range(D // 16):
            rows0[i, pl.ds(16 * k, 16)] = jnp.zeros((16,), jnp.float32)
        return 0
    lax.fori_loop(0, CHUNK, zfill, 0)
    base = sid * npr
    for t in range(npr // CHUNK):
        pltpu.sync_copy(rows0, acc.at[pl.ds(base + t * CHUNK, CHUNK)])
    plsc.subcore_barrier()

    def gather(j, buf, sem):
        for h in range(SPLIT):
            pltpu.async_copy(tbl.at[sidx_v.at[j, pl.ds(h * HS, HS)]],
                             buf.at[pl.ds(h * HS, HS)], sem)

    def gwait(j, buf, sem):
        for h in range(SPLIT):
            pltpu.make_async_copy(tbl.at[sidx_v.at[j, pl.ds(h * HS, HS)]],
                                  buf.at[pl.ds(h * HS, HS)], sem).wait()

    def group(g, _):
        pltpu.sync_copy(sidx.at[wid].at[pl.ds(g * G, G)], sidx_v)
        pltpu.sync_copy(didx.at[wid].at[pl.ds(g * G, G)], didx_v)

        gather(0, rows0, sem0)
        gather(1, rows1, sem1)

        def body(i, _):
            j0 = 2 * i
            gwait(j0, rows0, sem0)
            pltpu.sync_copy(rows0, acc.at[didx_v.at[j0]], add=True)

            @pl.when(i < G // 2 - 1)
            def _():
                gather(j0 + 2, rows0, sem0)

            gwait(j0 + 1, rows1, sem1)
            pltpu.sync_copy(rows1, acc.at[didx_v.at[j0 + 1]], add=True)

            @pl.when(i < G // 2 - 1)
            def _():
                gather(j0 + 3, rows1, sem1)
            return 0
        lax.fori_loop(0, G // 2, body, 0)
        return 0
    lax.fori_loop(0, NG, group, 0)
    plsc.subcore_barrier()

    pltpu.sync_copy(acc.at[pl.ds(base, npr)],
                    out.at[cid].at[pl.ds(base, npr)])


_segsum = pl.kernel(
    _segsum_body,
    out_type=jax.ShapeDtypeStruct((NC, NPAD, D), jnp.float32),
    mesh=_mesh(),
    scratch_types=[
        pltpu.VMEM((G, CHUNK), jnp.int32),
        pltpu.VMEM((G, CHUNK), jnp.int32),
        pltpu.VMEM((CHUNK, D), jnp.float32),
        pltpu.VMEM((CHUNK, D), jnp.float32),
        pltpu.VMEM_SHARED((NPAD, D), jnp.float32),
        pltpu.SemaphoreType.DMA,
        pltpu.SemaphoreType.DMA,
    ],
)


def _ones_scatter_body(didx, out, didx_v, rows0, acc):
    cid = lax.axis_index("c")
    sid = lax.axis_index("s")
    wid = cid * NS + sid
    npr = NPAD // NS

    def zfill(i, _):
        for k in range(D // 16):
            rows0[i, pl.ds(16 * k, 16)] = jnp.zeros((16,), jnp.float32)
        return 0
    lax.fori_loop(0, CHUNK, zfill, 0)
    base = sid * npr
    for t in range(npr // CHUNK):
        pltpu.sync_copy(rows0, acc.at[pl.ds(base + t * CHUNK, CHUNK)])

    def ofill(i, _):
        for k in range(D // 16):
            rows0[i, pl.ds(16 * k, 16)] = jnp.ones((16,), jnp.float32)
        return 0
    lax.fori_loop(0, CHUNK, ofill, 0)
    plsc.subcore_barrier()

    def group(g, _):
        pltpu.sync_copy(didx.at[wid].at[pl.ds(g * G, G)], didx_v)

        def body(j, _):
            pltpu.sync_copy(rows0, acc.at[didx_v.at[j]], add=True)
            return 0
        lax.fori_loop(0, G, body, 0)
        return 0
    lax.fori_loop(0, NG, group, 0)
    plsc.subcore_barrier()

    pltpu.sync_copy(acc.at[pl.ds(base, npr)],
                    out.at[cid].at[pl.ds(base, npr)])


_ones_scatter = pl.kernel(
    _ones_scatter_body,
    out_type=jax.ShapeDtypeStruct((NC, NPAD, D), jnp.float32),
    mesh=_mesh(),
    scratch_types=[
        pltpu.VMEM((G, CHUNK), jnp.int32),
        pltpu.VMEM((CHUNK, D), jnp.float32),
        pltpu.VMEM_SHARED((NPAD, D), jnp.float32),
    ],
)


def _dv_scale(dvp):
    cnt = dvp[0, :, 0] + dvp[1, :, 0]
    return jnp.where(cnt > 0, lax.rsqrt(cnt), 0.0)


def _de_scale(dep):
    cnt = dep[0, :, 0] + dep[1, :, 0]
    return jnp.where(cnt > 0, 1.0 / cnt, 0.0)


def _lrelu(x):
    return jnp.where(x >= 0, x, 0.1 * x)


def _stage_a_kernel(x_ref, w_ref, b_ref, dvp_ref, xs_ref, skip_ref):
    y = lax.dot_general(x_ref[...], w_ref[...], (((1,), (0,)), ((), ())),
                        preferred_element_type=jnp.float32) + b_ref[...]
    scale = _dv_scale(dvp_ref[...])
    xs_ref[...] = y[:, :D] * scale[:, None]
    skip_ref[...] = y[:, D:]


def _stage_a(x, w_cat, b_cat, dvp):
    nb = N_NODES // ROWS_BLK
    return pl.pallas_call(
        _stage_a_kernel,
        grid=(nb,),
        in_specs=[
            pl.BlockSpec((ROWS_BLK, D), lambda i: (i, 0)),
            pl.BlockSpec((D, 2 * D), lambda i: (0, 0)),
            pl.BlockSpec((1, 2 * D), lambda i: (0, 0)),
            pl.BlockSpec((NC, ROWS_BLK, D), lambda i: (0, i, 0)),
        ],
        out_specs=[pl.BlockSpec((ROWS_BLK, D), lambda i: (i, 0)),
                   pl.BlockSpec((ROWS_BLK, D), lambda i: (i, 0))],
        out_shape=[jax.ShapeDtypeStruct((NPAD, D), jnp.float32),
                   jax.ShapeDtypeStruct((N_NODES, D), jnp.float32)],
    )(x, w_cat, b_cat, dvp)


def _stage_b_kernel(ep_ref, dep_ref, eout_ref):
    de = _de_scale(dep_ref[...])[:, None]
    e = (ep_ref[0] + ep_ref[1]) * de
    eout_ref[...] = _lrelu(e) * de


def _stage_b(ep, dep):
    nb = N_EDGES // ROWS_BLK
    return pl.pallas_call(
        _stage_b_kernel,
        grid=(nb,),
        in_specs=[
            pl.BlockSpec((NC, ROWS_BLK, D), lambda i: (0, i, 0)),
            pl.BlockSpec((NC, ROWS_BLK, D), lambda i: (0, i, 0)),
        ],
        out_specs=pl.BlockSpec((ROWS_BLK, D), lambda i: (i, 0)),
        out_shape=jax.ShapeDtypeStruct((NPAD, D), jnp.float32),
    )(ep, dep)


def _stage_c_kernel(xp_ref, skip_ref, dvp_ref, xout_ref):
    dv = _dv_scale(dvp_ref[...])[:, None]
    xn = (xp_ref[0] + xp_ref[1]) * dv + skip_ref[...]
    xout_ref[...] = _lrelu(xn)


def _stage_c(xp, skip, dvp):
    nb = N_NODES // ROWS_BLK
    return pl.pallas_call(
        _stage_c_kernel,
        grid=(nb,),
        in_specs=[
            pl.BlockSpec((NC, ROWS_BLK, D), lambda i: (0, i, 0)),
            pl.BlockSpec((ROWS_BLK, D), lambda i: (i, 0)),
            pl.BlockSpec((NC, ROWS_BLK, D), lambda i: (0, i, 0)),
        ],
        out_specs=pl.BlockSpec((ROWS_BLK, D), lambda i: (i, 0)),
        out_shape=jax.ShapeDtypeStruct((N_NODES, D), jnp.float32),
    )(xp, skip, dvp)


def _scale_in_kernel(x_ref, dvp_ref, out_ref):
    out_ref[...] = x_ref[...] * _dv_scale(dvp_ref[...])[:, None]


def _scale_in(x, dvp):
    nb = N_NODES // ROWS_BLK
    return pl.pallas_call(
        _scale_in_kernel,
        grid=(nb,),
        in_specs=[
            pl.BlockSpec((ROWS_BLK, D), lambda i: (i, 0)),
            pl.BlockSpec((NC, ROWS_BLK, D), lambda i: (0, i, 0)),
        ],
        out_specs=pl.BlockSpec((ROWS_BLK, D), lambda i: (i, 0)),
        out_shape=jax.ShapeDtypeStruct((NPAD, D), jnp.float32),
    )(x, dvp)


def _final_e_kernel(ep_ref, dep_ref, out_ref):
    out_ref[...] = (ep_ref[0] + ep_ref[1]) * _de_scale(dep_ref[...])[:, None]


def _final_e(ep, dep):
    nb = N_EDGES // ROWS_BLK
    return pl.pallas_call(
        _final_e_kernel,
        grid=(nb,),
        in_specs=[
            pl.BlockSpec((NC, ROWS_BLK, D), lambda i: (0, i, 0)),
            pl.BlockSpec((NC, ROWS_BLK, D), lambda i: (0, i, 0)),
        ],
        out_specs=pl.BlockSpec((ROWS_BLK, D), lambda i: (i, 0)),
        out_shape=jax.ShapeDtypeStruct((N_EDGES, D), jnp.float32),
    )(ep, dep)


@jax.jit
def kernel(X, node_idx, edge_idx, params):
    pad = NNZ_PAD - NNZ
    shape3 = (NW, CPW, CHUNK)
    zpad = jnp.zeros((pad,), jnp.int32)
    gpad = jnp.full((pad,), GARBAGE, jnp.int32)
    nidx_src = jnp.concatenate([node_idx, zpad]).reshape(shape3)
    eidx_src = jnp.concatenate([edge_idx, zpad]).reshape(shape3)
    nidx_dst = jnp.concatenate([node_idx, gpad]).reshape(shape3)
    eidx_dst = jnp.concatenate([edge_idx, gpad]).reshape(shape3)

    dvp = _ones_scatter(nidx_dst)
    dep = _ones_scatter(eidx_dst)

    for layer in params:
        for wkey, bkey, pkey, pbkey in (("fc1_w", "fc1_b", "proj1_w", "proj1_b"),
                                        ("fc2_w", "fc2_b", "proj2_w", "proj2_b")):
            w_cat = jnp.concatenate(
                [layer[wkey].T, layer[pkey].T], axis=1)
            b_cat = jnp.concatenate(
                [layer[bkey], layer[pbkey]]).reshape(1, 2 * D)
            xs, skip = _stage_a(X, w_cat, b_cat, dvp)
            ep = _segsum(xs, nidx_src, eidx_dst)
            ein = _stage_b(ep, dep)
            xp = _segsum(ein, eidx_src, nidx_dst)
            X = _stage_c(xp, skip, dvp)

    xs_f = _scale_in(X, dvp)
    ep_f = _segsum(xs_f, nidx_src, eidx_dst)
    e_final = _final_e(ep_f, dep)
    return (e_final, X)


# speedup vs baseline: 1.2871x; 1.0196x over previous
import jax
import jax.numpy as jnp
from jax import lax
from jax.experimental import pallas as pl
from jax.experimental.pallas import tpu as pltpu
from jax.experimental.pallas import tpu_sc as plsc

N_NODES = 10000
N_EDGES = 5000
NNZ = 320000
D = 128

NC = 2
NS = 16
NW = NC * NS
CHUNK = 128
CPW = 80
G = 40
NG = CPW // G
NNZ_PAD = NW * CPW * CHUNK
NPAD = 10240
GARBAGE = NPAD - 1

SPLIT = 4
HS = CHUNK // SPLIT

ROWS_BLK = 1000


def _mesh():
    return plsc.VectorSubcoreMesh(core_axis_name="c", subcore_axis_name="s")


def _segsum_body(tbl, sidx, didx, out,
                 sidx_v, didx_v, rows0, rows1, acc, sem0, sem1):
    cid = lax.axis_index("c")
    sid = lax.axis_index("s")
    wid = cid * NS + sid
    npr = NPAD // NS

    def zfill(i, _):
        for k in range(D // 16):
            rows0[i, pl.ds(16 * k, 16)] = jnp.zeros((16,), jnp.float32)
        return 0
    lax.fori_loop(0, CHUNK, zfill, 0)
    base = sid * npr
    for t in range(npr // CHUNK):
        pltpu.sync_copy(rows0, acc.at[pl.ds(base + t * CHUNK, CHUNK)])
    plsc.subcore_barrier()

    def gather(j, buf, sem):
        for h in range(SPLIT):
            pltpu.async_copy(tbl.at[sidx_v.at[j, pl.ds(h * HS, HS)]],
                             buf.at[pl.ds(h * HS, HS)], sem)

    def gwait(j, buf, sem):
        for h in range(SPLIT):
            pltpu.make_async_copy(tbl.at[sidx_v.at[j, pl.ds(h * HS, HS)]],
                                  buf.at[pl.ds(h * HS, HS)], sem).wait()

    def group(g, _):
        pltpu.sync_copy(sidx.at[wid].at[pl.ds(g * G, G)], sidx_v)
        pltpu.sync_copy(didx.at[wid].at[pl.ds(g * G, G)], didx_v)

        gather(0, rows0, sem0)
        gather(1, rows1, sem1)

        def body(i, _):
            j0 = 2 * i
            gwait(j0, rows0, sem0)
            pltpu.sync_copy(rows0, acc.at[didx_v.at[j0]], add=True)

            @pl.when(i < G // 2 - 1)
            def _():
                gather(j0 + 2, rows0, sem0)

            gwait(j0 + 1, rows1, sem1)
            pltpu.sync_copy(rows1, acc.at[didx_v.at[j0 + 1]], add=True)

            @pl.when(i < G // 2 - 1)
            def _():
                gather(j0 + 3, rows1, sem1)
            return 0
        lax.fori_loop(0, G // 2, body, 0)
        return 0
    lax.fori_loop(0, NG, group, 0)
    plsc.subcore_barrier()

    pltpu.sync_copy(acc.at[pl.ds(base, npr)],
                    out.at[cid].at[pl.ds(base, npr)])


_segsum = pl.kernel(
    _segsum_body,
    out_type=jax.ShapeDtypeStruct((NC, NPAD, D), jnp.float32),
    mesh=_mesh(),
    scratch_types=[
        pltpu.VMEM((G, CHUNK), jnp.int32),
        pltpu.VMEM((G, CHUNK), jnp.int32),
        pltpu.VMEM((CHUNK, D), jnp.float32),
        pltpu.VMEM((CHUNK, D), jnp.float32),
        pltpu.VMEM_SHARED((NPAD, D), jnp.float32),
        pltpu.SemaphoreType.DMA,
        pltpu.SemaphoreType.DMA,
    ],
)


def _ones_scatter_body(didx, out, didx_v, rows0, acc):
    cid = lax.axis_index("c")
    sid = lax.axis_index("s")
    wid = cid * NS + sid
    npr = NPAD // NS

    def zfill(i, _):
        for k in range(D // 16):
            rows0[i, pl.ds(16 * k, 16)] = jnp.zeros((16,), jnp.float32)
        return 0
    lax.fori_loop(0, CHUNK, zfill, 0)
    base = sid * npr
    for t in range(npr // CHUNK):
        pltpu.sync_copy(rows0, acc.at[pl.ds(base + t * CHUNK, CHUNK)])

    def ofill(i, _):
        for k in range(D // 16):
            rows0[i, pl.ds(16 * k, 16)] = jnp.ones((16,), jnp.float32)
        return 0
    lax.fori_loop(0, CHUNK, ofill, 0)
    plsc.subcore_barrier()

    def group(g, _):
        pltpu.sync_copy(didx.at[wid].at[pl.ds(g * G, G)], didx_v)

        def body(j, _):
            pltpu.sync_copy(rows0, acc.at[didx_v.at[j]], add=True)
            return 0
        lax.fori_loop(0, G, body, 0)
        return 0
    lax.fori_loop(0, NG, group, 0)
    plsc.subcore_barrier()

    pltpu.sync_copy(acc.at[pl.ds(base, npr)],
                    out.at[cid].at[pl.ds(base, npr)])


_ones_scatter = pl.kernel(
    _ones_scatter_body,
    out_type=jax.ShapeDtypeStruct((NC, NPAD, D), jnp.float32),
    mesh=_mesh(),
    scratch_types=[
        pltpu.VMEM((G, CHUNK), jnp.int32),
        pltpu.VMEM((CHUNK, D), jnp.float32),
        pltpu.VMEM_SHARED((NPAD, D), jnp.float32),
    ],
)


def _dv_scale(dvp):
    cnt = dvp[0, :, 0] + dvp[1, :, 0]
    return jnp.where(cnt > 0, lax.rsqrt(cnt), 0.0)


def _de_scale(dep):
    cnt = dep[0, :, 0] + dep[1, :, 0]
    return jnp.where(cnt > 0, 1.0 / cnt, 0.0)


def _lrelu(x):
    return jnp.where(x >= 0, x, 0.1 * x)


def _stage_a_kernel(x_ref, w_ref, b_ref, dvp_ref, xs_ref, skip_ref):
    y = lax.dot_general(x_ref[...], w_ref[...], (((1,), (0,)), ((), ())),
                        preferred_element_type=jnp.float32) + b_ref[...]
    scale = _dv_scale(dvp_ref[...])
    xs_ref[...] = y[:, :D] * scale[:, None]
    skip_ref[...] = y[:, D:]


def _stage_a(x, w_cat, b_cat, dvp):
    nb = N_NODES // ROWS_BLK
    return pl.pallas_call(
        _stage_a_kernel,
        grid=(nb,),
        in_specs=[
            pl.BlockSpec((ROWS_BLK, D), lambda i: (i, 0)),
            pl.BlockSpec((D, 2 * D), lambda i: (0, 0)),
            pl.BlockSpec((1, 2 * D), lambda i: (0, 0)),
            pl.BlockSpec((NC, ROWS_BLK, D), lambda i: (0, i, 0)),
        ],
        out_specs=[pl.BlockSpec((ROWS_BLK, D), lambda i: (i, 0)),
                   pl.BlockSpec((ROWS_BLK, D), lambda i: (i, 0))],
        out_shape=[jax.ShapeDtypeStruct((NPAD, D), jnp.float32),
                   jax.ShapeDtypeStruct((N_NODES, D), jnp.float32)],
    )(x, w_cat, b_cat, dvp)


def _stage_b_kernel(ep_ref, dep_ref, eout_ref):
    de = _de_scale(dep_ref[...])[:, None]
    e = (ep_ref[0] + ep_ref[1]) * de
    eout_ref[...] = _lrelu(e) * de


def _stage_b(ep, dep):
    nb = N_EDGES // ROWS_BLK
    return pl.pallas_call(
        _stage_b_kernel,
        grid=(nb,),
        in_specs=[
            pl.BlockSpec((NC, ROWS_BLK, D), lambda i: (0, i, 0)),
            pl.BlockSpec((NC, ROWS_BLK, D), lambda i: (0, i, 0)),
        ],
        out_specs=pl.BlockSpec((ROWS_BLK, D), lambda i: (i, 0)),
        out_shape=jax.ShapeDtypeStruct((NPAD, D), jnp.float32),
    )(ep, dep)


def _stage_c_kernel(xp_ref, skip_ref, dvp_ref, xout_ref):
    dv = _dv_scale(dvp_ref[...])[:, None]
    xn = (xp_ref[0] + xp_ref[1]) * dv + skip_ref[...]
    xout_ref[...] = _lrelu(xn)


def _stage_c(xp, skip, dvp):
    nb = N_NODES // ROWS_BLK
    return pl.pallas_call(
        _stage_c_kernel,
        grid=(nb,),
        in_specs=[
            pl.BlockSpec((NC, ROWS_BLK, D), lambda i: (0, i, 0)),
            pl.BlockSpec((ROWS_BLK, D), lambda i: (i, 0)),
            pl.BlockSpec((NC, ROWS_BLK, D), lambda i: (0, i, 0)),
        ],
        out_specs=pl.BlockSpec((ROWS_BLK, D), lambda i: (i, 0)),
        out_shape=jax.ShapeDtypeStruct((N_NODES, D), jnp.float32),
    )(xp, skip, dvp)


def _scale_in_kernel(x_ref, dvp_ref, out_ref):
    out_ref[...] = x_ref[...] * _dv_scale(dvp_ref[...])[:, None]


def _scale_in(x, dvp):
    nb = N_NODES // ROWS_BLK
    return pl.pallas_call(
        _scale_in_kernel,
        grid=(nb,),
        in_specs=[
            pl.BlockSpec((ROWS_BLK, D), lambda i: (i, 0)),
            pl.BlockSpec((NC, ROWS_BLK, D), lambda i: (0, i, 0)),
        ],
        out_specs=pl.BlockSpec((ROWS_BLK, D), lambda i: (i, 0)),
        out_shape=jax.ShapeDtypeStruct((NPAD, D), jnp.float32),
    )(x, dvp)


def _final_e_kernel(ep_ref, dep_ref, out_ref):
    out_ref[...] = (ep_ref[0] + ep_ref[1]) * _de_scale(dep_ref[...])[:, None]


def _final_e(ep, dep):
    nb = N_EDGES // ROWS_BLK
    return pl.pallas_call(
        _final_e_kernel,
        grid=(nb,),
        in_specs=[
            pl.BlockSpec((NC, ROWS_BLK, D), lambda i: (0, i, 0)),
            pl.BlockSpec((NC, ROWS_BLK, D), lambda i: (0, i, 0)),
        ],
        out_specs=pl.BlockSpec((ROWS_BLK, D), lambda i: (i, 0)),
        out_shape=jax.ShapeDtypeStruct((N_EDGES, D), jnp.float32),
    )(ep, dep)


@jax.jit
def kernel(X, node_idx, edge_idx, params):
    pad = NNZ_PAD - NNZ
    shape3 = (NW, CPW, CHUNK)
    zpad = jnp.zeros((pad,), jnp.int32)
    gpad = jnp.full((pad,), GARBAGE, jnp.int32)
    nidx_src = jnp.concatenate([node_idx, zpad]).reshape(shape3)
    eidx_src = jnp.concatenate([edge_idx, zpad]).reshape(shape3)
    nidx_dst = jnp.concatenate([node_idx, gpad]).reshape(shape3)
    eidx_dst = jnp.concatenate([edge_idx, gpad]).reshape(shape3)

    dvp = _ones_scatter(nidx_dst)
    dep = _ones_scatter(eidx_dst)

    for layer in params:
        for wkey, bkey, pkey, pbkey in (("fc1_w", "fc1_b", "proj1_w", "proj1_b"),
                                        ("fc2_w", "fc2_b", "proj2_w", "proj2_b")):
            w_cat = jnp.concatenate(
                [layer[wkey].T, layer[pkey].T], axis=1)
            b_cat = jnp.concatenate(
                [layer[bkey], layer[pbkey]]).reshape(1, 2 * D)
            xs, skip = _stage_a(X, w_cat, b_cat, dvp)
            ep = _segsum(xs, nidx_src, eidx_dst)
            ein = _stage_b(ep, dep)
            xp = _segsum(ein, eidx_src, nidx_dst)
            X = _stage_c(xp, skip, dvp)

    xs_f = _scale_in(X, dvp)
    ep_f = _segsum(xs_f, nidx_src, eidx_dst)
    e_final = _final_e(ep_f, dep)
    return (e_final, X)


# speedup vs baseline: 1.3012x; 1.0109x over previous
import jax
import jax.numpy as jnp
from jax import lax
from jax.experimental import pallas as pl
from jax.experimental.pallas import tpu as pltpu
from jax.experimental.pallas import tpu_sc as plsc

N_NODES = 10000
N_EDGES = 5000
NNZ = 320000
D = 128

NC = 2
NS = 16
NW = NC * NS
CHUNK = 128
CPW = 80
G = 40
NG = CPW // G
NNZ_PAD = NW * CPW * CHUNK
NPAD = 10240
EPAD = 5120
GARBAGE = NPAD - 1
EGARBAGE = EPAD - 1

SPLIT = 4
HS = CHUNK // SPLIT

ROWS_BLK = 1000


def _mesh():
    return plsc.VectorSubcoreMesh(core_axis_name="c", subcore_axis_name="s")


def _make_segsum(ndst_pad, nbuf, grp):
    ngrp = CPW // grp
    npr = ndst_pad // NS

    def body_fn(tbl, sidx, didx, out, *refs):
        sidx_v, didx_v = refs[0], refs[1]
        rows = refs[2:2 + nbuf]
        acc = refs[2 + nbuf]
        sems = refs[3 + nbuf:3 + 2 * nbuf]
        cid = lax.axis_index("c")
        sid = lax.axis_index("s")
        wid = cid * NS + sid

        def zfill(i, _):
            for k in range(D // 16):
                rows[0][i, pl.ds(16 * k, 16)] = jnp.zeros((16,), jnp.float32)
            return 0
        lax.fori_loop(0, CHUNK, zfill, 0)
        base = sid * npr
        for t in range(npr // CHUNK):
            pltpu.sync_copy(rows[0], acc.at[pl.ds(base + t * CHUNK, CHUNK)])
        rem = npr % CHUNK
        if rem:
            pltpu.sync_copy(rows[0].at[pl.ds(0, rem)],
                            acc.at[pl.ds(base + npr - rem, rem)])
        plsc.subcore_barrier()

        def gather(j, buf, sem):
            for h in range(SPLIT):
                pltpu.async_copy(tbl.at[sidx_v.at[j, pl.ds(h * HS, HS)]],
                                 buf.at[pl.ds(h * HS, HS)], sem)

        def gwait(j, buf, sem):
            for h in range(SPLIT):
                pltpu.make_async_copy(tbl.at[sidx_v.at[j, pl.ds(h * HS, HS)]],
                                      buf.at[pl.ds(h * HS, HS)], sem).wait()

        def group(g, _):
            pltpu.sync_copy(sidx.at[wid].at[pl.ds(g * grp, grp)], sidx_v)
            pltpu.sync_copy(didx.at[wid].at[pl.ds(g * grp, grp)], didx_v)

            for b in range(nbuf):
                gather(b, rows[b], sems[b])

            def body(i, _):
                j0 = nbuf * i
                for b in range(nbuf):
                    gwait(j0 + b, rows[b], sems[b])
                    pltpu.sync_copy(rows[b], acc.at[didx_v.at[j0 + b]],
                                    add=True)

                    @pl.when(i < grp // nbuf - 1)
                    def _():
                        gather(j0 + nbuf + b, rows[b], sems[b])
                return 0
            lax.fori_loop(0, grp // nbuf, body, 0)
            return 0
        lax.fori_loop(0, ngrp, group, 0)
        plsc.subcore_barrier()

        pltpu.sync_copy(acc.at[pl.ds(base, npr)],
                        out.at[cid].at[pl.ds(base, npr)])

    return pl.kernel(
        body_fn,
        out_type=jax.ShapeDtypeStruct((NC, ndst_pad, D), jnp.float32),
        mesh=_mesh(),
        scratch_types=(
            [pltpu.VMEM((grp, CHUNK), jnp.int32),
             pltpu.VMEM((grp, CHUNK), jnp.int32)]
            + [pltpu.VMEM((CHUNK, D), jnp.float32) for _ in range(nbuf)]
            + [pltpu.VMEM_SHARED((ndst_pad, D), jnp.float32)]
            + [pltpu.SemaphoreType.DMA for _ in range(nbuf)]
        ),
    )


_segsum_node = _make_segsum(NPAD, 2, 40)
_segsum_edge = _make_segsum(EPAD, 4, 40)

def _ones_scatter_body(didx, out, didx_v, rows0, acc):
    cid = lax.axis_index("c")
    sid = lax.axis_index("s")
    wid = cid * NS + sid
    npr = NPAD // NS

    def zfill(i, _):
        for k in range(D // 16):
            rows0[i, pl.ds(16 * k, 16)] = jnp.zeros((16,), jnp.float32)
        return 0
    lax.fori_loop(0, CHUNK, zfill, 0)
    base = sid * npr
    for t in range(npr // CHUNK):
        pltpu.sync_copy(rows0, acc.at[pl.ds(base + t * CHUNK, CHUNK)])

    def ofill(i, _):
        for k in range(D // 16):
            rows0[i, pl.ds(16 * k, 16)] = jnp.ones((16,), jnp.float32)
        return 0
    lax.fori_loop(0, CHUNK, ofill, 0)
    plsc.subcore_barrier()

    def group(g, _):
        pltpu.sync_copy(didx.at[wid].at[pl.ds(g * G, G)], didx_v)

        def body(j, _):
            pltpu.sync_copy(rows0, acc.at[didx_v.at[j]], add=True)
            return 0
        lax.fori_loop(0, G, body, 0)
        return 0
    lax.fori_loop(0, NG, group, 0)
    plsc.subcore_barrier()

    pltpu.sync_copy(acc.at[pl.ds(base, npr)],
                    out.at[cid].at[pl.ds(base, npr)])


_ones_scatter = pl.kernel(
    _ones_scatter_body,
    out_type=jax.ShapeDtypeStruct((NC, NPAD, D), jnp.float32),
    mesh=_mesh(),
    scratch_types=[
        pltpu.VMEM((G, CHUNK), jnp.int32),
        pltpu.VMEM((CHUNK, D), jnp.float32),
        pltpu.VMEM_SHARED((NPAD, D), jnp.float32),
    ],
)


def _dv_scale(dvp):
    cnt = dvp[0, :, 0] + dvp[1, :, 0]
    return jnp.where(cnt > 0, lax.rsqrt(cnt), 0.0)


def _de_scale(dep):
    cnt = dep[0, :, 0] + dep[1, :, 0]
    return jnp.where(cnt > 0, 1.0 / cnt, 0.0)


def _lrelu(x):
    return jnp.where(x >= 0, x, 0.1 * x)


def _stage_a_kernel(x_ref, w_ref, b_ref, dvp_ref, xs_ref, skip_ref):
    y = lax.dot_general(x_ref[...], w_ref[...], (((1,), (0,)), ((), ())),
                        preferred_element_type=jnp.float32) + b_ref[...]
    scale = _dv_scale(dvp_ref[...])
    xs_ref[...] = y[:, :D] * scale[:, None]
    skip_ref[...] = y[:, D:]


def _stage_a(x, w_cat, b_cat, dvp):
    nb = N_NODES // ROWS_BLK
    return pl.pallas_call(
        _stage_a_kernel,
        grid=(nb,),
        in_specs=[
            pl.BlockSpec((ROWS_BLK, D), lambda i: (i, 0)),
            pl.BlockSpec((D, 2 * D), lambda i: (0, 0)),
            pl.BlockSpec((1, 2 * D), lambda i: (0, 0)),
            pl.BlockSpec((NC, ROWS_BLK, D), lambda i: (0, i, 0)),
        ],
        out_specs=[pl.BlockSpec((ROWS_BLK, D), lambda i: (i, 0)),
                   pl.BlockSpec((ROWS_BLK, D), lambda i: (i, 0))],
        out_shape=[jax.ShapeDtypeStruct((NPAD, D), jnp.float32),
                   jax.ShapeDtypeStruct((N_NODES, D), jnp.float32)],
    )(x, w_cat, b_cat, dvp)


def _stage_b_kernel(ep_ref, dep_ref, eout_ref):
    de = _de_scale(dep_ref[...])[:, None]
    e = (ep_ref[0] + ep_ref[1]) * de
    eout_ref[...] = _lrelu(e) * de


def _stage_b(ep, dep):
    nb = N_EDGES // ROWS_BLK
    return pl.pallas_call(
        _stage_b_kernel,
        grid=(nb,),
        in_specs=[
            pl.BlockSpec((NC, ROWS_BLK, D), lambda i: (0, i, 0)),
            pl.BlockSpec((NC, ROWS_BLK, D), lambda i: (0, i, 0)),
        ],
        out_specs=pl.BlockSpec((ROWS_BLK, D), lambda i: (i, 0)),
        out_shape=jax.ShapeDtypeStruct((NPAD, D), jnp.float32),
    )(ep, dep)


def _stage_c_kernel(xp_ref, skip_ref, dvp_ref, xout_ref):
    dv = _dv_scale(dvp_ref[...])[:, None]
    xn = (xp_ref[0] + xp_ref[1]) * dv + skip_ref[...]
    xout_ref[...] = _lrelu(xn)


def _stage_c(xp, skip, dvp):
    nb = N_NODES // ROWS_BLK
    return pl.pallas_call(
        _stage_c_kernel,
        grid=(nb,),
        in_specs=[
            pl.BlockSpec((NC, ROWS_BLK, D), lambda i: (0, i, 0)),
            pl.BlockSpec((ROWS_BLK, D), lambda i: (i, 0)),
            pl.BlockSpec((NC, ROWS_BLK, D), lambda i: (0, i, 0)),
        ],
        out_specs=pl.BlockSpec((ROWS_BLK, D), lambda i: (i, 0)),
        out_shape=jax.ShapeDtypeStruct((N_NODES, D), jnp.float32),
    )(xp, skip, dvp)


def _scale_in_kernel(x_ref, dvp_ref, out_ref):
    out_ref[...] = x_ref[...] * _dv_scale(dvp_ref[...])[:, None]


def _scale_in(x, dvp):
    nb = N_NODES // ROWS_BLK
    return pl.pallas_call(
        _scale_in_kernel,
        grid=(nb,),
        in_specs=[
            pl.BlockSpec((ROWS_BLK, D), lambda i: (i, 0)),
            pl.BlockSpec((NC, ROWS_BLK, D), lambda i: (0, i, 0)),
        ],
        out_specs=pl.BlockSpec((ROWS_BLK, D), lambda i: (i, 0)),
        out_shape=jax.ShapeDtypeStruct((NPAD, D), jnp.float32),
    )(x, dvp)


def _final_e_kernel(ep_ref, dep_ref, out_ref):
    out_ref[...] = (ep_ref[0] + ep_ref[1]) * _de_scale(dep_ref[...])[:, None]


def _final_e(ep, dep):
    nb = N_EDGES // ROWS_BLK
    return pl.pallas_call(
        _final_e_kernel,
        grid=(nb,),
        in_specs=[
            pl.BlockSpec((NC, ROWS_BLK, D), lambda i: (0, i, 0)),
            pl.BlockSpec((NC, ROWS_BLK, D), lambda i: (0, i, 0)),
        ],
        out_specs=pl.BlockSpec((ROWS_BLK, D), lambda i: (i, 0)),
        out_shape=jax.ShapeDtypeStruct((N_EDGES, D), jnp.float32),
    )(ep, dep)


@jax.jit
def kernel(X, node_idx, edge_idx, params):
    pad = NNZ_PAD - NNZ
    shape3 = (NW, CPW, CHUNK)
    zpad = jnp.zeros((pad,), jnp.int32)
    gpad = jnp.full((pad,), GARBAGE, jnp.int32)
    egpad = jnp.full((pad,), EGARBAGE, jnp.int32)
    nidx_src = jnp.concatenate([node_idx, zpad]).reshape(shape3)
    eidx_src = jnp.concatenate([edge_idx, zpad]).reshape(shape3)
    nidx_dst = jnp.concatenate([node_idx, gpad]).reshape(shape3)
    eidx_dst = jnp.concatenate([edge_idx, egpad]).reshape(shape3)

    dvp = _ones_scatter(nidx_dst)
    dep = _ones_scatter(eidx_dst)

    for layer in params:
        for wkey, bkey, pkey, pbkey in (("fc1_w", "fc1_b", "proj1_w", "proj1_b"),
                                        ("fc2_w", "fc2_b", "proj2_w", "proj2_b")):
            w_cat = jnp.concatenate(
                [layer[wkey].T, layer[pkey].T], axis=1)
            b_cat = jnp.concatenate(
                [layer[bkey], layer[pbkey]]).reshape(1, 2 * D)
            xs, skip = _stage_a(X, w_cat, b_cat, dvp)
            ep = _segsum_edge(xs, nidx_src, eidx_dst)
            ein = _stage_b(ep, dep)
            xp = _segsum_node(ein, eidx_src, nidx_dst)
            X = _stage_c(xp, skip, dvp)

    xs_f = _scale_in(X, dvp)
    ep_f = _segsum_edge(xs_f, nidx_src, eidx_dst)
    e_final = _final_e(ep_f, dep)
    return (e_final, X)


# speedup vs baseline: 2.0546x; 1.5790x over previous
import jax
import jax.numpy as jnp
from jax import lax
from jax.experimental import pallas as pl
from jax.experimental.pallas import tpu as pltpu
from jax.experimental.pallas import tpu_sc as plsc

N_NODES = 10000
N_EDGES = 5000
NNZ = 320000
D = 128

NC = 2
NS = 16
NW = NC * NS
CHUNK = 128
CPW = 80
G = 40
NG = CPW // G
NNZ_PAD = NW * CPW * CHUNK
NPAD = 10240
EPAD = 5120
XREP = 2
EREP = 4
GARBAGE = NPAD - 1
EGARBAGE = EPAD - 1

SPLIT = 4
HS = CHUNK // SPLIT

ROWS_BLK = 1000


def _mesh():
    return plsc.VectorSubcoreMesh(core_axis_name="c", subcore_axis_name="s")


def _make_segsum(ndst_pad, nbuf, grp):
    ngrp = CPW // grp
    npr = ndst_pad // NS

    def body_fn(tbl, sidx, didx, out, *refs):
        sidx_v, didx_v = refs[0], refs[1]
        rows = refs[2:2 + nbuf]
        acc = refs[2 + nbuf]
        sems = refs[3 + nbuf:3 + 2 * nbuf]
        cid = lax.axis_index("c")
        sid = lax.axis_index("s")
        wid = cid * NS + sid

        def zfill(i, _):
            for k in range(D // 16):
                rows[0][i, pl.ds(16 * k, 16)] = jnp.zeros((16,), jnp.float32)
            return 0
        lax.fori_loop(0, CHUNK, zfill, 0)
        base = sid * npr
        for t in range(npr // CHUNK):
            pltpu.sync_copy(rows[0], acc.at[pl.ds(base + t * CHUNK, CHUNK)])
        rem = npr % CHUNK
        if rem:
            pltpu.sync_copy(rows[0].at[pl.ds(0, rem)],
                            acc.at[pl.ds(base + npr - rem, rem)])
        plsc.subcore_barrier()

        def gather(j, buf, sem):
            for h in range(SPLIT):
                pltpu.async_copy(tbl.at[sidx_v.at[j, pl.ds(h * HS, HS)]],
                                 buf.at[pl.ds(h * HS, HS)], sem)

        def gwait(j, buf, sem):
            for h in range(SPLIT):
                pltpu.make_async_copy(tbl.at[sidx_v.at[j, pl.ds(h * HS, HS)]],
                                      buf.at[pl.ds(h * HS, HS)], sem).wait()

        def group(g, _):
            pltpu.sync_copy(sidx.at[wid].at[pl.ds(g * grp, grp)], sidx_v)
            pltpu.sync_copy(didx.at[wid].at[pl.ds(g * grp, grp)], didx_v)

            for b in range(nbuf):
                gather(b, rows[b], sems[b])

            def body(i, _):
                j0 = nbuf * i
                for b in range(nbuf):
                    gwait(j0 + b, rows[b], sems[b])
                    pltpu.sync_copy(rows[b], acc.at[didx_v.at[j0 + b]],
                                    add=True)

                    @pl.when(i < grp // nbuf - 1)
                    def _():
                        gather(j0 + nbuf + b, rows[b], sems[b])
                return 0
            lax.fori_loop(0, grp // nbuf, body, 0)
            return 0
        lax.fori_loop(0, ngrp, group, 0)
        plsc.subcore_barrier()

        pltpu.sync_copy(acc.at[pl.ds(base, npr)],
                        out.at[cid].at[pl.ds(base, npr)])

    return pl.kernel(
        body_fn,
        out_type=jax.ShapeDtypeStruct((NC, ndst_pad, D), jnp.float32),
        mesh=_mesh(),
        scratch_types=(
            [pltpu.VMEM((grp, CHUNK), jnp.int32),
             pltpu.VMEM((grp, CHUNK), jnp.int32)]
            + [pltpu.VMEM((CHUNK, D), jnp.float32) for _ in range(nbuf)]
            + [pltpu.VMEM_SHARED((ndst_pad, D), jnp.float32)]
            + [pltpu.SemaphoreType.DMA for _ in range(nbuf)]
        ),
    )


_segsum_node = _make_segsum(NPAD, 2, 40)
_segsum_edge = _make_segsum(EPAD, 4, 40)

def _ones_scatter_body(didx, out, didx_v, rows0, acc):
    cid = lax.axis_index("c")
    sid = lax.axis_index("s")
    wid = cid * NS + sid
    npr = NPAD // NS

    def zfill(i, _):
        for k in range(D // 16):
            rows0[i, pl.ds(16 * k, 16)] = jnp.zeros((16,), jnp.float32)
        return 0
    lax.fori_loop(0, CHUNK, zfill, 0)
    base = sid * npr
    for t in range(npr // CHUNK):
        pltpu.sync_copy(rows0, acc.at[pl.ds(base + t * CHUNK, CHUNK)])

    def ofill(i, _):
        for k in range(D // 16):
            rows0[i, pl.ds(16 * k, 16)] = jnp.ones((16,), jnp.float32)
        return 0
    lax.fori_loop(0, CHUNK, ofill, 0)
    plsc.subcore_barrier()

    def group(g, _):
        pltpu.sync_copy(didx.at[wid].at[pl.ds(g * G, G)], didx_v)

        def body(j, _):
            pltpu.sync_copy(rows0, acc.at[didx_v.at[j]], add=True)
            return 0
        lax.fori_loop(0, G, body, 0)
        return 0
    lax.fori_loop(0, NG, group, 0)
    plsc.subcore_barrier()

    pltpu.sync_copy(acc.at[pl.ds(base, npr)],
                    out.at[cid].at[pl.ds(base, npr)])


_ones_scatter = pl.kernel(
    _ones_scatter_body,
    out_type=jax.ShapeDtypeStruct((NC, NPAD, D), jnp.float32),
    mesh=_mesh(),
    scratch_types=[
        pltpu.VMEM((G, CHUNK), jnp.int32),
        pltpu.VMEM((CHUNK, D), jnp.float32),
        pltpu.VMEM_SHARED((NPAD, D), jnp.float32),
    ],
)


def _dv_scale(dvp):
    cnt = dvp[0, :, 0] + dvp[1, :, 0]
    return jnp.where(cnt > 0, lax.rsqrt(cnt), 0.0)


def _de_scale(dep):
    cnt = dep[0, :, 0] + dep[1, :, 0]
    return jnp.where(cnt > 0, 1.0 / cnt, 0.0)


def _lrelu(x):
    return jnp.where(x >= 0, x, 0.1 * x)


def _stage_a_kernel(x_ref, w_ref, b_ref, dvp_ref, xs_ref, skip_ref):
    y = lax.dot_general(x_ref[...], w_ref[...], (((1,), (0,)), ((), ())),
                        preferred_element_type=jnp.float32) + b_ref[...]
    scale = _dv_scale(dvp_ref[...])
    xs_ref[...] = y[:, :D] * scale[:, None]
    skip_ref[...] = y[:, D:]


def _stage_a_kernel_rep(x_ref, w_ref, b_ref, dvp_ref, xs_ref, skip_ref):
    y = lax.dot_general(x_ref[...], w_ref[...], (((1,), (0,)), ((), ())),
                        preferred_element_type=jnp.float32) + b_ref[...]
    scale = _dv_scale(dvp_ref[...])
    xs_ref[0] = y[:, :D] * scale[:, None]
    skip_ref[...] = y[:, D:]


def _stage_a(x, w_cat, b_cat, dvp):
    nb = N_NODES // ROWS_BLK
    return pl.pallas_call(
        _stage_a_kernel_rep,
        grid=(XREP, nb),
        in_specs=[
            pl.BlockSpec((ROWS_BLK, D), lambda j, i: (i, 0)),
            pl.BlockSpec((D, 2 * D), lambda j, i: (0, 0)),
            pl.BlockSpec((1, 2 * D), lambda j, i: (0, 0)),
            pl.BlockSpec((NC, ROWS_BLK, D), lambda j, i: (0, i, 0)),
        ],
        out_specs=[pl.BlockSpec((1, ROWS_BLK, D), lambda j, i: (j, i, 0)),
                   pl.BlockSpec((ROWS_BLK, D), lambda j, i: (i, 0))],
        out_shape=[jax.ShapeDtypeStruct((XREP, NPAD, D), jnp.float32),
                   jax.ShapeDtypeStruct((N_NODES, D), jnp.float32)],
    )(x, w_cat, b_cat, dvp)


def _stage_b_kernel(ep_ref, dep_ref, eout_ref):
    de = _de_scale(dep_ref[...])[:, None]
    e = (ep_ref[0] + ep_ref[1]) * de
    eout_ref[0] = _lrelu(e) * de


def _stage_b(ep, dep):
    nb = N_EDGES // ROWS_BLK
    return pl.pallas_call(
        _stage_b_kernel,
        grid=(EREP, nb),
        in_specs=[
            pl.BlockSpec((NC, ROWS_BLK, D), lambda j, i: (0, i, 0)),
            pl.BlockSpec((NC, ROWS_BLK, D), lambda j, i: (0, i, 0)),
        ],
        out_specs=pl.BlockSpec((1, ROWS_BLK, D), lambda j, i: (j, i, 0)),
        out_shape=jax.ShapeDtypeStruct((EREP, EPAD, D), jnp.float32),
    )(ep, dep)


def _stage_c_kernel(xp_ref, skip_ref, dvp_ref, xout_ref):
    dv = _dv_scale(dvp_ref[...])[:, None]
    xn = (xp_ref[0] + xp_ref[1]) * dv + skip_ref[...]
    xout_ref[...] = _lrelu(xn)


def _stage_c(xp, skip, dvp):
    nb = N_NODES // ROWS_BLK
    return pl.pallas_call(
        _stage_c_kernel,
        grid=(nb,),
        in_specs=[
            pl.BlockSpec((NC, ROWS_BLK, D), lambda i: (0, i, 0)),
            pl.BlockSpec((ROWS_BLK, D), lambda i: (i, 0)),
            pl.BlockSpec((NC, ROWS_BLK, D), lambda i: (0, i, 0)),
        ],
        out_specs=pl.BlockSpec((ROWS_BLK, D), lambda i: (i, 0)),
        out_shape=jax.ShapeDtypeStruct((N_NODES, D), jnp.float32),
    )(xp, skip, dvp)


def _scale_in_kernel(x_ref, dvp_ref, out_ref):
    out_ref[0] = x_ref[...] * _dv_scale(dvp_ref[...])[:, None]


def _scale_in(x, dvp):
    nb = N_NODES // ROWS_BLK
    return pl.pallas_call(
        _scale_in_kernel,
        grid=(XREP, nb),
        in_specs=[
            pl.BlockSpec((ROWS_BLK, D), lambda j, i: (i, 0)),
            pl.BlockSpec((NC, ROWS_BLK, D), lambda j, i: (0, i, 0)),
        ],
        out_specs=pl.BlockSpec((1, ROWS_BLK, D), lambda j, i: (j, i, 0)),
        out_shape=jax.ShapeDtypeStruct((XREP, NPAD, D), jnp.float32),
    )(x, dvp)


def _final_e_kernel(ep_ref, dep_ref, out_ref):
    out_ref[...] = (ep_ref[0] + ep_ref[1]) * _de_scale(dep_ref[...])[:, None]


def _final_e(ep, dep):
    nb = N_EDGES // ROWS_BLK
    return pl.pallas_call(
        _final_e_kernel,
        grid=(nb,),
        in_specs=[
            pl.BlockSpec((NC, ROWS_BLK, D), lambda i: (0, i, 0)),
            pl.BlockSpec((NC, ROWS_BLK, D), lambda i: (0, i, 0)),
        ],
        out_specs=pl.BlockSpec((ROWS_BLK, D), lambda i: (i, 0)),
        out_shape=jax.ShapeDtypeStruct((N_EDGES, D), jnp.float32),
    )(ep, dep)


@jax.jit
def kernel(X, node_idx, edge_idx, params):
    pad = NNZ_PAD - NNZ
    shape3 = (NW, CPW, CHUNK)
    zpad = jnp.zeros((pad,), jnp.int32)
    gpad = jnp.full((pad,), GARBAGE, jnp.int32)
    egpad = jnp.full((pad,), EGARBAGE, jnp.int32)
    xrep_off = (jnp.arange(NNZ_PAD, dtype=jnp.int32) % XREP) * NPAD
    erep_off = (jnp.arange(NNZ_PAD, dtype=jnp.int32) % EREP) * EPAD
    nidx_src = (jnp.concatenate([node_idx, zpad]) + xrep_off).reshape(shape3)
    eidx_src = (jnp.concatenate([edge_idx, zpad]) + erep_off).reshape(shape3)
    nidx_dst = jnp.concatenate([node_idx, gpad]).reshape(shape3)
    eidx_dst = jnp.concatenate([edge_idx, egpad]).reshape(shape3)

    dvp = _ones_scatter(nidx_dst)
    dep = _ones_scatter(eidx_dst)

    for layer in params:
        for wkey, bkey, pkey, pbkey in (("fc1_w", "fc1_b", "proj1_w", "proj1_b"),
                                        ("fc2_w", "fc2_b", "proj2_w", "proj2_b")):
            w_cat = jnp.concatenate(
                [layer[wkey].T, layer[pkey].T], axis=1)
            b_cat = jnp.concatenate(
                [layer[bkey], layer[pbkey]]).reshape(1, 2 * D)
            xs, skip = _stage_a(X, w_cat, b_cat, dvp)
            ep = _segsum_edge(xs.reshape(XREP * NPAD, D), nidx_src, eidx_dst)
            ein = _stage_b(ep, dep)
            xp = _segsum_node(ein.reshape(EREP * EPAD, D), eidx_src, nidx_dst)
            X = _stage_c(xp, skip, dvp)

    xs_f = _scale_in(X, dvp)
    ep_f = _segsum_edge(xs_f.reshape(XREP * NPAD, D), nidx_src, eidx_dst)
    e_final = _final_e(ep_f, dep)
    return (e_final, X)


# speedup vs baseline: 2.2824x; 1.1109x over previous
import jax
import jax.numpy as jnp
from jax import lax
from jax.experimental import pallas as pl
from jax.experimental.pallas import tpu as pltpu
from jax.experimental.pallas import tpu_sc as plsc

N_NODES = 10000
N_EDGES = 5000
NNZ = 320000
D = 128

NC = 2
NS = 16
NW = NC * NS
CHUNK = 128
CPW = 80
G = 40
NG = CPW // G
NNZ_PAD = NW * CPW * CHUNK
NPAD = 10240
EPAD = 5120
XREP = 4
EREP = 8
GARBAGE = NPAD - 1
EGARBAGE = EPAD - 1

SPLIT = 4
HS = CHUNK // SPLIT

ROWS_BLK = 1000


def _mesh():
    return plsc.VectorSubcoreMesh(core_axis_name="c", subcore_axis_name="s")


def _make_segsum(ndst_pad, nbuf, grp):
    ngrp = CPW // grp
    npr = ndst_pad // NS

    def body_fn(tbl, sidx, didx, out, *refs):
        sidx_v, didx_v = refs[0], refs[1]
        rows = refs[2:2 + nbuf]
        acc = refs[2 + nbuf]
        sems = refs[3 + nbuf:3 + 2 * nbuf]
        cid = lax.axis_index("c")
        sid = lax.axis_index("s")
        wid = cid * NS + sid

        def zfill(i, _):
            for k in range(D // 16):
                rows[0][i, pl.ds(16 * k, 16)] = jnp.zeros((16,), jnp.float32)
            return 0
        lax.fori_loop(0, CHUNK, zfill, 0)
        base = sid * npr
        for t in range(npr // CHUNK):
            pltpu.sync_copy(rows[0], acc.at[pl.ds(base + t * CHUNK, CHUNK)])
        rem = npr % CHUNK
        if rem:
            pltpu.sync_copy(rows[0].at[pl.ds(0, rem)],
                            acc.at[pl.ds(base + npr - rem, rem)])
        plsc.subcore_barrier()

        def gather(j, buf, sem):
            for h in range(SPLIT):
                pltpu.async_copy(tbl.at[sidx_v.at[j, pl.ds(h * HS, HS)]],
                                 buf.at[pl.ds(h * HS, HS)], sem)

        def gwait(j, buf, sem):
            for h in range(SPLIT):
                pltpu.make_async_copy(tbl.at[sidx_v.at[j, pl.ds(h * HS, HS)]],
                                      buf.at[pl.ds(h * HS, HS)], sem).wait()

        def group(g, _):
            pltpu.sync_copy(sidx.at[wid].at[pl.ds(g * grp, grp)], sidx_v)
            pltpu.sync_copy(didx.at[wid].at[pl.ds(g * grp, grp)], didx_v)

            for b in range(nbuf):
                gather(b, rows[b], sems[b])

            def body(i, _):
                j0 = nbuf * i
                for b in range(nbuf):
                    gwait(j0 + b, rows[b], sems[b])
                    pltpu.sync_copy(rows[b], acc.at[didx_v.at[j0 + b]],
                                    add=True)

                    @pl.when(i < grp // nbuf - 1)
                    def _():
                        gather(j0 + nbuf + b, rows[b], sems[b])
                return 0
            lax.fori_loop(0, grp // nbuf, body, 0)
            return 0
        lax.fori_loop(0, ngrp, group, 0)
        plsc.subcore_barrier()

        pltpu.sync_copy(acc.at[pl.ds(base, npr)],
                        out.at[cid].at[pl.ds(base, npr)])

    return pl.kernel(
        body_fn,
        out_type=jax.ShapeDtypeStruct((NC, ndst_pad, D), jnp.float32),
        mesh=_mesh(),
        scratch_types=(
            [pltpu.VMEM((grp, CHUNK), jnp.int32),
             pltpu.VMEM((grp, CHUNK), jnp.int32)]
            + [pltpu.VMEM((CHUNK, D), jnp.float32) for _ in range(nbuf)]
            + [pltpu.VMEM_SHARED((ndst_pad, D), jnp.float32)]
            + [pltpu.SemaphoreType.DMA for _ in range(nbuf)]
        ),
    )


_segsum_node = _make_segsum(NPAD, 2, 40)
_segsum_edge = _make_segsum(EPAD, 4, 40)

def _ones_scatter_body(didx, out, didx_v, rows0, acc):
    cid = lax.axis_index("c")
    sid = lax.axis_index("s")
    wid = cid * NS + sid
    npr = NPAD // NS

    def zfill(i, _):
        for k in range(D // 16):
            rows0[i, pl.ds(16 * k, 16)] = jnp.zeros((16,), jnp.float32)
        return 0
    lax.fori_loop(0, CHUNK, zfill, 0)
    base = sid * npr
    for t in range(npr // CHUNK):
        pltpu.sync_copy(rows0, acc.at[pl.ds(base + t * CHUNK, CHUNK)])

    def ofill(i, _):
        for k in range(D // 16):
            rows0[i, pl.ds(16 * k, 16)] = jnp.ones((16,), jnp.float32)
        return 0
    lax.fori_loop(0, CHUNK, ofill, 0)
    plsc.subcore_barrier()

    def group(g, _):
        pltpu.sync_copy(didx.at[wid].at[pl.ds(g * G, G)], didx_v)

        def body(j, _):
            pltpu.sync_copy(rows0, acc.at[didx_v.at[j]], add=True)
            return 0
        lax.fori_loop(0, G, body, 0)
        return 0
    lax.fori_loop(0, NG, group, 0)
    plsc.subcore_barrier()

    pltpu.sync_copy(acc.at[pl.ds(base, npr)],
                    out.at[cid].at[pl.ds(base, npr)])


_ones_scatter = pl.kernel(
    _ones_scatter_body,
    out_type=jax.ShapeDtypeStruct((NC, NPAD, D), jnp.float32),
    mesh=_mesh(),
    scratch_types=[
        pltpu.VMEM((G, CHUNK), jnp.int32),
        pltpu.VMEM((CHUNK, D), jnp.float32),
        pltpu.VMEM_SHARED((NPAD, D), jnp.float32),
    ],
)


def _dv_scale(dvp):
    cnt = dvp[0, :, 0] + dvp[1, :, 0]
    return jnp.where(cnt > 0, lax.rsqrt(cnt), 0.0)


def _de_scale(dep):
    cnt = dep[0, :, 0] + dep[1, :, 0]
    return jnp.where(cnt > 0, 1.0 / cnt, 0.0)


def _lrelu(x):
    return jnp.where(x >= 0, x, 0.1 * x)


def _stage_a_kernel(x_ref, w_ref, b_ref, dvp_ref, xs_ref, skip_ref):
    y = lax.dot_general(x_ref[...], w_ref[...], (((1,), (0,)), ((), ())),
                        preferred_element_type=jnp.float32) + b_ref[...]
    scale = _dv_scale(dvp_ref[...])
    xs_ref[...] = y[:, :D] * scale[:, None]
    skip_ref[...] = y[:, D:]


def _stage_a_kernel_rep(x_ref, w_ref, b_ref, dvp_ref, xs_ref, skip_ref):
    y = lax.dot_general(x_ref[...], w_ref[...], (((1,), (0,)), ((), ())),
                        preferred_element_type=jnp.float32) + b_ref[...]
    scale = _dv_scale(dvp_ref[...])
    xs_ref[0] = y[:, :D] * scale[:, None]
    skip_ref[...] = y[:, D:]


def _stage_a(x, w_cat, b_cat, dvp):
    nb = N_NODES // ROWS_BLK
    return pl.pallas_call(
        _stage_a_kernel_rep,
        grid=(XREP, nb),
        in_specs=[
            pl.BlockSpec((ROWS_BLK, D), lambda j, i: (i, 0)),
            pl.BlockSpec((D, 2 * D), lambda j, i: (0, 0)),
            pl.BlockSpec((1, 2 * D), lambda j, i: (0, 0)),
            pl.BlockSpec((NC, ROWS_BLK, D), lambda j, i: (0, i, 0)),
        ],
        out_specs=[pl.BlockSpec((1, ROWS_BLK, D), lambda j, i: (j, i, 0)),
                   pl.BlockSpec((ROWS_BLK, D), lambda j, i: (i, 0))],
        out_shape=[jax.ShapeDtypeStruct((XREP, NPAD, D), jnp.float32),
                   jax.ShapeDtypeStruct((N_NODES, D), jnp.float32)],
    )(x, w_cat, b_cat, dvp)


def _stage_b_kernel(ep_ref, dep_ref, eout_ref):
    de = _de_scale(dep_ref[...])[:, None]
    e = (ep_ref[0] + ep_ref[1]) * de
    eout_ref[0] = _lrelu(e) * de


def _stage_b(ep, dep):
    nb = N_EDGES // ROWS_BLK
    return pl.pallas_call(
        _stage_b_kernel,
        grid=(EREP, nb),
        in_specs=[
            pl.BlockSpec((NC, ROWS_BLK, D), lambda j, i: (0, i, 0)),
            pl.BlockSpec((NC, ROWS_BLK, D), lambda j, i: (0, i, 0)),
        ],
        out_specs=pl.BlockSpec((1, ROWS_BLK, D), lambda j, i: (j, i, 0)),
        out_shape=jax.ShapeDtypeStruct((EREP, EPAD, D), jnp.float32),
    )(ep, dep)


def _stage_c_kernel(xp_ref, skip_ref, dvp_ref, xout_ref):
    dv = _dv_scale(dvp_ref[...])[:, None]
    xn = (xp_ref[0] + xp_ref[1]) * dv + skip_ref[...]
    xout_ref[...] = _lrelu(xn)


def _stage_c(xp, skip, dvp):
    nb = N_NODES // ROWS_BLK
    return pl.pallas_call(
        _stage_c_kernel,
        grid=(nb,),
        in_specs=[
            pl.BlockSpec((NC, ROWS_BLK, D), lambda i: (0, i, 0)),
            pl.BlockSpec((ROWS_BLK, D), lambda i: (i, 0)),
            pl.BlockSpec((NC, ROWS_BLK, D), lambda i: (0, i, 0)),
        ],
        out_specs=pl.BlockSpec((ROWS_BLK, D), lambda i: (i, 0)),
        out_shape=jax.ShapeDtypeStruct((N_NODES, D), jnp.float32),
    )(xp, skip, dvp)


def _scale_in_kernel(x_ref, dvp_ref, out_ref):
    out_ref[0] = x_ref[...] * _dv_scale(dvp_ref[...])[:, None]


def _scale_in(x, dvp):
    nb = N_NODES // ROWS_BLK
    return pl.pallas_call(
        _scale_in_kernel,
        grid=(XREP, nb),
        in_specs=[
            pl.BlockSpec((ROWS_BLK, D), lambda j, i: (i, 0)),
            pl.BlockSpec((NC, ROWS_BLK, D), lambda j, i: (0, i, 0)),
        ],
        out_specs=pl.BlockSpec((1, ROWS_BLK, D), lambda j, i: (j, i, 0)),
        out_shape=jax.ShapeDtypeStruct((XREP, NPAD, D), jnp.float32),
    )(x, dvp)


def _final_e_kernel(ep_ref, dep_ref, out_ref):
    out_ref[...] = (ep_ref[0] + ep_ref[1]) * _de_scale(dep_ref[...])[:, None]


def _final_e(ep, dep):
    nb = N_EDGES // ROWS_BLK
    return pl.pallas_call(
        _final_e_kernel,
        grid=(nb,),
        in_specs=[
            pl.BlockSpec((NC, ROWS_BLK, D), lambda i: (0, i, 0)),
            pl.BlockSpec((NC, ROWS_BLK, D), lambda i: (0, i, 0)),
        ],
        out_specs=pl.BlockSpec((ROWS_BLK, D), lambda i: (i, 0)),
        out_shape=jax.ShapeDtypeStruct((N_EDGES, D), jnp.float32),
    )(ep, dep)


@jax.jit
def kernel(X, node_idx, edge_idx, params):
    pad = NNZ_PAD - NNZ
    shape3 = (NW, CPW, CHUNK)
    zpad = jnp.zeros((pad,), jnp.int32)
    gpad = jnp.full((pad,), GARBAGE, jnp.int32)
    egpad = jnp.full((pad,), EGARBAGE, jnp.int32)
    xrep_off = (jnp.arange(NNZ_PAD, dtype=jnp.int32) % XREP) * NPAD
    erep_off = (jnp.arange(NNZ_PAD, dtype=jnp.int32) % EREP) * EPAD
    nidx_src = (jnp.concatenate([node_idx, zpad]) + xrep_off).reshape(shape3)
    eidx_src = (jnp.concatenate([edge_idx, zpad]) + erep_off).reshape(shape3)
    nidx_dst = jnp.concatenate([node_idx, gpad]).reshape(shape3)
    eidx_dst = jnp.concatenate([edge_idx, egpad]).reshape(shape3)

    dvp = _ones_scatter(nidx_dst)
    dep = _ones_scatter(eidx_dst)

    for layer in params:
        for wkey, bkey, pkey, pbkey in (("fc1_w", "fc1_b", "proj1_w", "proj1_b"),
                                        ("fc2_w", "fc2_b", "proj2_w", "proj2_b")):
            w_cat = jnp.concatenate(
                [layer[wkey].T, layer[pkey].T], axis=1)
            b_cat = jnp.concatenate(
                [layer[bkey], layer[pbkey]]).reshape(1, 2 * D)
            xs, skip = _stage_a(X, w_cat, b_cat, dvp)
            ep = _segsum_edge(xs.reshape(XREP * NPAD, D), nidx_src, eidx_dst)
            ein = _stage_b(ep, dep)
            xp = _segsum_node(ein.reshape(EREP * EPAD, D), eidx_src, nidx_dst)
            X = _stage_c(xp, skip, dvp)

    xs_f = _scale_in(X, dvp)
    ep_f = _segsum_edge(xs_f.reshape(XREP * NPAD, D), nidx_src, eidx_dst)
    e_final = _final_e(ep_f, dep)
    return (e_final, X)


# speedup vs baseline: 2.4467x; 1.0720x over previous
import jax
import jax.numpy as jnp
from jax import lax
from jax.experimental import pallas as pl
from jax.experimental.pallas import tpu as pltpu
from jax.experimental.pallas import tpu_sc as plsc

N_NODES = 10000
N_EDGES = 5000
NNZ = 320000
D = 128

NC = 2
NS = 16
NW = NC * NS
CHUNK = 128
CPW = 80
G = 40
NG = CPW // G
NNZ_PAD = NW * CPW * CHUNK
NPAD = 10240
EPAD = 5120
XREP = 8
EREP = 16
GARBAGE = NPAD - 1
EGARBAGE = EPAD - 1

SPLIT = 4
HS = CHUNK // SPLIT

ROWS_BLK = 1000


def _mesh():
    return plsc.VectorSubcoreMesh(core_axis_name="c", subcore_axis_name="s")


def _make_segsum(ndst_pad, nbuf, grp):
    ngrp = CPW // grp
    npr = ndst_pad // NS

    def body_fn(tbl, sidx, didx, out, *refs):
        sidx_v, didx_v = refs[0], refs[1]
        rows = refs[2:2 + nbuf]
        acc = refs[2 + nbuf]
        sems = refs[3 + nbuf:3 + 2 * nbuf]
        cid = lax.axis_index("c")
        sid = lax.axis_index("s")
        wid = cid * NS + sid

        def zfill(i, _):
            for k in range(D // 16):
                rows[0][i, pl.ds(16 * k, 16)] = jnp.zeros((16,), jnp.float32)
            return 0
        lax.fori_loop(0, CHUNK, zfill, 0)
        base = sid * npr
        for t in range(npr // CHUNK):
            pltpu.sync_copy(rows[0], acc.at[pl.ds(base + t * CHUNK, CHUNK)])
        rem = npr % CHUNK
        if rem:
            pltpu.sync_copy(rows[0].at[pl.ds(0, rem)],
                            acc.at[pl.ds(base + npr - rem, rem)])
        plsc.subcore_barrier()

        def gather(j, buf, sem):
            for h in range(SPLIT):
                pltpu.async_copy(tbl.at[sidx_v.at[j, pl.ds(h * HS, HS)]],
                                 buf.at[pl.ds(h * HS, HS)], sem)

        def gwait(j, buf, sem):
            for h in range(SPLIT):
                pltpu.make_async_copy(tbl.at[sidx_v.at[j, pl.ds(h * HS, HS)]],
                                      buf.at[pl.ds(h * HS, HS)], sem).wait()

        def group(g, _):
            pltpu.sync_copy(sidx.at[wid].at[pl.ds(g * grp, grp)], sidx_v)
            pltpu.sync_copy(didx.at[wid].at[pl.ds(g * grp, grp)], didx_v)

            for b in range(nbuf):
                gather(b, rows[b], sems[b])

            def body(i, _):
                j0 = nbuf * i
                for b in range(nbuf):
                    gwait(j0 + b, rows[b], sems[b])
                    pltpu.sync_copy(rows[b], acc.at[didx_v.at[j0 + b]],
                                    add=True)

                    @pl.when(i < grp // nbuf - 1)
                    def _():
                        gather(j0 + nbuf + b, rows[b], sems[b])
                return 0
            lax.fori_loop(0, grp // nbuf, body, 0)
            return 0
        lax.fori_loop(0, ngrp, group, 0)
        plsc.subcore_barrier()

        pltpu.sync_copy(acc.at[pl.ds(base, npr)],
                        out.at[cid].at[pl.ds(base, npr)])

    return pl.kernel(
        body_fn,
        out_type=jax.ShapeDtypeStruct((NC, ndst_pad, D), jnp.float32),
        mesh=_mesh(),
        scratch_types=(
            [pltpu.VMEM((grp, CHUNK), jnp.int32),
             pltpu.VMEM((grp, CHUNK), jnp.int32)]
            + [pltpu.VMEM((CHUNK, D), jnp.float32) for _ in range(nbuf)]
            + [pltpu.VMEM_SHARED((ndst_pad, D), jnp.float32)]
            + [pltpu.SemaphoreType.DMA for _ in range(nbuf)]
        ),
    )


_segsum_node = _make_segsum(NPAD, 2, 40)
_segsum_edge = _make_segsum(EPAD, 4, 40)

def _ones_scatter_body(didx, out, didx_v, rows0, acc):
    cid = lax.axis_index("c")
    sid = lax.axis_index("s")
    wid = cid * NS + sid
    npr = NPAD // NS

    def zfill(i, _):
        for k in range(D // 16):
            rows0[i, pl.ds(16 * k, 16)] = jnp.zeros((16,), jnp.float32)
        return 0
    lax.fori_loop(0, CHUNK, zfill, 0)
    base = sid * npr
    for t in range(npr // CHUNK):
        pltpu.sync_copy(rows0, acc.at[pl.ds(base + t * CHUNK, CHUNK)])

    def ofill(i, _):
        for k in range(D // 16):
            rows0[i, pl.ds(16 * k, 16)] = jnp.ones((16,), jnp.float32)
        return 0
    lax.fori_loop(0, CHUNK, ofill, 0)
    plsc.subcore_barrier()

    def group(g, _):
        pltpu.sync_copy(didx.at[wid].at[pl.ds(g * G, G)], didx_v)

        def body(j, _):
            pltpu.sync_copy(rows0, acc.at[didx_v.at[j]], add=True)
            return 0
        lax.fori_loop(0, G, body, 0)
        return 0
    lax.fori_loop(0, NG, group, 0)
    plsc.subcore_barrier()

    pltpu.sync_copy(acc.at[pl.ds(base, npr)],
                    out.at[cid].at[pl.ds(base, npr)])


_ones_scatter = pl.kernel(
    _ones_scatter_body,
    out_type=jax.ShapeDtypeStruct((NC, NPAD, D), jnp.float32),
    mesh=_mesh(),
    scratch_types=[
        pltpu.VMEM((G, CHUNK), jnp.int32),
        pltpu.VMEM((CHUNK, D), jnp.float32),
        pltpu.VMEM_SHARED((NPAD, D), jnp.float32),
    ],
)


def _dv_scale(dvp):
    cnt = dvp[0, :, 0] + dvp[1, :, 0]
    return jnp.where(cnt > 0, lax.rsqrt(cnt), 0.0)


def _de_scale(dep):
    cnt = dep[0, :, 0] + dep[1, :, 0]
    return jnp.where(cnt > 0, 1.0 / cnt, 0.0)


def _lrelu(x):
    return jnp.where(x >= 0, x, 0.1 * x)


def _stage_a_kernel(x_ref, w_ref, b_ref, dvp_ref, xs_ref, skip_ref):
    y = lax.dot_general(x_ref[...], w_ref[...], (((1,), (0,)), ((), ())),
                        preferred_element_type=jnp.float32) + b_ref[...]
    scale = _dv_scale(dvp_ref[...])
    xs_ref[...] = y[:, :D] * scale[:, None]
    skip_ref[...] = y[:, D:]


def _stage_a_kernel_rep(x_ref, w_ref, b_ref, dvp_ref, xs_ref, skip_ref):
    y = lax.dot_general(x_ref[...], w_ref[...], (((1,), (0,)), ((), ())),
                        preferred_element_type=jnp.float32) + b_ref[...]
    scale = _dv_scale(dvp_ref[...])
    xs_ref[0] = y[:, :D] * scale[:, None]
    skip_ref[...] = y[:, D:]


def _stage_a(x, w_cat, b_cat, dvp):
    nb = N_NODES // ROWS_BLK
    return pl.pallas_call(
        _stage_a_kernel_rep,
        grid=(XREP, nb),
        in_specs=[
            pl.BlockSpec((ROWS_BLK, D), lambda j, i: (i, 0)),
            pl.BlockSpec((D, 2 * D), lambda j, i: (0, 0)),
            pl.BlockSpec((1, 2 * D), lambda j, i: (0, 0)),
            pl.BlockSpec((NC, ROWS_BLK, D), lambda j, i: (0, i, 0)),
        ],
        out_specs=[pl.BlockSpec((1, ROWS_BLK, D), lambda j, i: (j, i, 0)),
                   pl.BlockSpec((ROWS_BLK, D), lambda j, i: (i, 0))],
        out_shape=[jax.ShapeDtypeStruct((XREP, NPAD, D), jnp.float32),
                   jax.ShapeDtypeStruct((N_NODES, D), jnp.float32)],
    )(x, w_cat, b_cat, dvp)


def _stage_b_kernel(ep_ref, dep_ref, eout_ref):
    de = _de_scale(dep_ref[...])[:, None]
    e = (ep_ref[0] + ep_ref[1]) * de
    eout_ref[0] = _lrelu(e) * de


def _stage_b(ep, dep):
    nb = N_EDGES // ROWS_BLK
    return pl.pallas_call(
        _stage_b_kernel,
        grid=(EREP, nb),
        in_specs=[
            pl.BlockSpec((NC, ROWS_BLK, D), lambda j, i: (0, i, 0)),
            pl.BlockSpec((NC, ROWS_BLK, D), lambda j, i: (0, i, 0)),
        ],
        out_specs=pl.BlockSpec((1, ROWS_BLK, D), lambda j, i: (j, i, 0)),
        out_shape=jax.ShapeDtypeStruct((EREP, EPAD, D), jnp.float32),
    )(ep, dep)


def _stage_c_kernel(xp_ref, skip_ref, dvp_ref, xout_ref):
    dv = _dv_scale(dvp_ref[...])[:, None]
    xn = (xp_ref[0] + xp_ref[1]) * dv + skip_ref[...]
    xout_ref[...] = _lrelu(xn)


def _stage_c(xp, skip, dvp):
    nb = N_NODES // ROWS_BLK
    return pl.pallas_call(
        _stage_c_kernel,
        grid=(nb,),
        in_specs=[
            pl.BlockSpec((NC, ROWS_BLK, D), lambda i: (0, i, 0)),
            pl.BlockSpec((ROWS_BLK, D), lambda i: (i, 0)),
            pl.BlockSpec((NC, ROWS_BLK, D), lambda i: (0, i, 0)),
        ],
        out_specs=pl.BlockSpec((ROWS_BLK, D), lambda i: (i, 0)),
        out_shape=jax.ShapeDtypeStruct((N_NODES, D), jnp.float32),
    )(xp, skip, dvp)


def _scale_in_kernel(x_ref, dvp_ref, out_ref):
    out_ref[0] = x_ref[...] * _dv_scale(dvp_ref[...])[:, None]


def _scale_in(x, dvp):
    nb = N_NODES // ROWS_BLK
    return pl.pallas_call(
        _scale_in_kernel,
        grid=(XREP, nb),
        in_specs=[
            pl.BlockSpec((ROWS_BLK, D), lambda j, i: (i, 0)),
            pl.BlockSpec((NC, ROWS_BLK, D), lambda j, i: (0, i, 0)),
        ],
        out_specs=pl.BlockSpec((1, ROWS_BLK, D), lambda j, i: (j, i, 0)),
        out_shape=jax.ShapeDtypeStruct((XREP, NPAD, D), jnp.float32),
    )(x, dvp)


def _final_e_kernel(ep_ref, dep_ref, out_ref):
    out_ref[...] = (ep_ref[0] + ep_ref[1]) * _de_scale(dep_ref[...])[:, None]


def _final_e(ep, dep):
    nb = N_EDGES // ROWS_BLK
    return pl.pallas_call(
        _final_e_kernel,
        grid=(nb,),
        in_specs=[
            pl.BlockSpec((NC, ROWS_BLK, D), lambda i: (0, i, 0)),
            pl.BlockSpec((NC, ROWS_BLK, D), lambda i: (0, i, 0)),
        ],
        out_specs=pl.BlockSpec((ROWS_BLK, D), lambda i: (i, 0)),
        out_shape=jax.ShapeDtypeStruct((N_EDGES, D), jnp.float32),
    )(ep, dep)


@jax.jit
def kernel(X, node_idx, edge_idx, params):
    pad = NNZ_PAD - NNZ
    shape3 = (NW, CPW, CHUNK)
    zpad = jnp.zeros((pad,), jnp.int32)
    gpad = jnp.full((pad,), GARBAGE, jnp.int32)
    egpad = jnp.full((pad,), EGARBAGE, jnp.int32)
    xrep_off = (jnp.arange(NNZ_PAD, dtype=jnp.int32) % XREP) * NPAD
    erep_off = (jnp.arange(NNZ_PAD, dtype=jnp.int32) % EREP) * EPAD
    nidx_src = (jnp.concatenate([node_idx, zpad]) + xrep_off).reshape(shape3)
    eidx_src = (jnp.concatenate([edge_idx, zpad]) + erep_off).reshape(shape3)
    nidx_dst = jnp.concatenate([node_idx, gpad]).reshape(shape3)
    eidx_dst = jnp.concatenate([edge_idx, egpad]).reshape(shape3)

    dvp = _ones_scatter(nidx_dst)
    dep = _ones_scatter(eidx_dst)

    for layer in params:
        for wkey, bkey, pkey, pbkey in (("fc1_w", "fc1_b", "proj1_w", "proj1_b"),
                                        ("fc2_w", "fc2_b", "proj2_w", "proj2_b")):
            w_cat = jnp.concatenate(
                [layer[wkey].T, layer[pkey].T], axis=1)
            b_cat = jnp.concatenate(
                [layer[bkey], layer[pbkey]]).reshape(1, 2 * D)
            xs, skip = _stage_a(X, w_cat, b_cat, dvp)
            ep = _segsum_edge(xs.reshape(XREP * NPAD, D), nidx_src, eidx_dst)
            ein = _stage_b(ep, dep)
            xp = _segsum_node(ein.reshape(EREP * EPAD, D), eidx_src, nidx_dst)
            X = _stage_c(xp, skip, dvp)

    xs_f = _scale_in(X, dvp)
    ep_f = _segsum_edge(xs_f.reshape(XREP * NPAD, D), nidx_src, eidx_dst)
    e_final = _final_e(ep_f, dep)
    return (e_final, X)
